# baseline (speedup 1.0000x reference)
import jax
import jax.numpy as jnp
from jax.experimental import pallas as pl
from jax.experimental.pallas import tpu as pltpu

_BM = 1024
_NBUF = 6
_CS = 2


def _mlp_fused(x_hbm, w1_ref, b1_ref, wp_ref, bp_ref, out_ref, xbuf, sem):
    i = pl.program_id(0)
    nblk = pl.num_programs(0)
    d = x_hbm.shape[1]
    cw = d // _CS

    def copy(blk, slot, c):
        return pltpu.make_async_copy(
            x_hbm.at[pl.ds(blk * _BM, _BM), pl.ds(c * cw, cw)],
            xbuf.at[slot, :, pl.ds(c * cw, cw)],
            sem.at[slot, c],
        )

    @pl.when(i == 0)
    def _prologue():
        for s in range(_NBUF):
            for c in range(_CS):
                copy(s, s, c).start()

    slot = jax.lax.rem(i, _NBUF)
    for c in range(_CS):
        copy(i, slot, c).wait()

    x = xbuf[slot]
    feats = jnp.tanh(
        jnp.dot(x, w1_ref[...], preferred_element_type=jnp.float32)
        + b1_ref[...]
    )
    out_ref[...] = (
        jnp.dot(feats, wp_ref[...], preferred_element_type=jnp.float32)
        + bp_ref[...]
    )

    @pl.when(i + _NBUF < nblk)
    def _refill():
        for c in range(_CS):
            copy(i + _NBUF, slot, c).start()


def kernel(inputs, W1, b1, W2, b2, Wp, bp):
    B, D = inputs.shape
    H = W1.shape[1]
    O = Wp.shape[1]
    b1r = b1.reshape(1, H)
    bpr = bp.reshape(1, O)
    out = pl.pallas_call(
        _mlp_fused,
        grid=(B // _BM,),
        in_specs=[
            pl.BlockSpec(memory_space=pltpu.MemorySpace.HBM),
            pl.BlockSpec((D, H), lambda i: (0, 0)),
            pl.BlockSpec((1, H), lambda i: (0, 0)),
            pl.BlockSpec((D, O), lambda i: (0, 0)),
            pl.BlockSpec((1, O), lambda i: (0, 0)),
        ],
        out_specs=pl.BlockSpec((_BM, O), lambda i: (i, 0)),
        out_shape=jax.ShapeDtypeStruct((B, O), jnp.float32),
        scratch_shapes=[
            pltpu.VMEM((_NBUF, _BM, 512), jnp.float32),
            pltpu.SemaphoreType.DMA((_NBUF, _CS)),
        ],
        compiler_params=pltpu.CompilerParams(
            dimension_semantics=("arbitrary",),
        ),
    )(inputs, W1, b1r, Wp, bpr)
    return out


# speedup vs baseline: 1.1078x; 1.1078x over previous
import jax
import jax.numpy as jnp
from jax.experimental import pallas as pl
from jax.experimental.pallas import tpu as pltpu

_XS = 4
_BMS = 1024


def _mlp_fused(x0, x1, x2, x3, w1_ref, b1_ref, wp_ref, bp_ref, out_ref):
    for k, xr in enumerate((x0, x1, x2, x3)):
        feats = jnp.tanh(
            jnp.dot(xr[...], w1_ref[...], preferred_element_type=jnp.float32)
            + b1_ref[...]
        )
        out_ref[pl.ds(k * _BMS, _BMS), :] = (
            jnp.dot(feats, wp_ref[...], preferred_element_type=jnp.float32)
            + bp_ref[...]
        )


def kernel(inputs, W1, b1, W2, b2, Wp, bp):
    B, D = inputs.shape
    H = W1.shape[1]
    O = Wp.shape[1]
    bm = _XS * _BMS
    b1r = b1.reshape(1, H)
    bpr = bp.reshape(1, O)

    def xspec(k):
        return pl.BlockSpec((_BMS, D), lambda i, k=k: (_XS * i + k, 0))

    out = pl.pallas_call(
        _mlp_fused,
        grid=(B // bm,),
        in_specs=[
            xspec(0), xspec(1), xspec(2), xspec(3),
            pl.BlockSpec((D, H), lambda i: (0, 0)),
            pl.BlockSpec((1, H), lambda i: (0, 0)),
            pl.BlockSpec((D, O), lambda i: (0, 0)),
            pl.BlockSpec((1, O), lambda i: (0, 0)),
        ],
        out_specs=pl.BlockSpec((bm, O), lambda i: (i, 0)),
        out_shape=jax.ShapeDtypeStruct((B, O), jnp.float32),
        compiler_params=pltpu.CompilerParams(
            dimension_semantics=("arbitrary",),
        ),
    )(inputs, inputs, inputs, inputs, W1, b1r, Wp, bpr)
    return out


# speedup vs baseline: 1.5955x; 1.4402x over previous
import jax
import jax.numpy as jnp
from jax.experimental import pallas as pl
from jax.experimental.pallas import tpu as pltpu

_XS = 4
_BMS = 1024


def _mlp_fused(x0, x1, x2, x3, w1_ref, b1_ref, wpt_ref, bp_ref, out_ref):
    bpt = jnp.transpose(bp_ref[...])
    for k, xr in enumerate((x0, x1, x2, x3)):
        feats = jnp.tanh(
            jnp.dot(xr[...], w1_ref[...], preferred_element_type=jnp.float32)
            + b1_ref[...]
        )
        out_t = jax.lax.dot_general(
            wpt_ref[...], feats,
            (((1,), (1,)), ((), ())),
            preferred_element_type=jnp.float32,
        )
        out_ref[:, pl.ds(k * _BMS, _BMS)] = out_t + bpt


def kernel(inputs, W1, b1, W2, b2, Wp, bp):
    B, D = inputs.shape
    H = W1.shape[1]
    O = Wp.shape[1]
    bm = _XS * _BMS
    b1r = b1.reshape(1, H)
    wpt = Wp.T
    bpr = bp.reshape(1, O)

    def xspec(k):
        return pl.BlockSpec((_BMS, D), lambda i, k=k: (_XS * i + k, 0))

    out_t = pl.pallas_call(
        _mlp_fused,
        grid=(B // bm,),
        in_specs=[
            xspec(0), xspec(1), xspec(2), xspec(3),
            pl.BlockSpec((D, H), lambda i: (0, 0)),
            pl.BlockSpec((1, H), lambda i: (0, 0)),
            pl.BlockSpec((O, D), lambda i: (0, 0)),
            pl.BlockSpec((1, O), lambda i: (0, 0)),
        ],
        out_specs=pl.BlockSpec((O, bm), lambda i: (0, i)),
        out_shape=jax.ShapeDtypeStruct((O, B), jnp.float32),
        compiler_params=pltpu.CompilerParams(
            dimension_semantics=("arbitrary",),
        ),
    )(inputs, inputs, inputs, inputs, W1, b1r, wpt, bpr)
    return out_t.T
